# speedup vs baseline: 2.7605x; 2.7605x over previous
import jax
import jax.numpy as jnp
from jax import lax
from jax.experimental import pallas as pl
from jax.experimental.pallas import tpu as pltpu
from jax.experimental.pallas import tpu_sc as plsc

_NC = 2
_NS = 16
_NW = _NC * _NS
_CHUNK = 128


def _body(idx_hbm, tab_hbm, out_hbm, idx_v, rows_v, sem):
    wid = lax.axis_index("s") * _NC + lax.axis_index("c")
    nchunk = idx_v.shape[0]
    pltpu.sync_copy(idx_hbm.at[wid], idx_v)
    row0 = pl.multiple_of(wid * (nchunk * _CHUNK), _CHUNK)

    @pl.loop(0, nchunk)
    def _step(j):
        pltpu.async_copy(tab_hbm.at[idx_v.at[j]], rows_v, sem).wait()
        pltpu.sync_copy(rows_v, out_hbm.at[pl.ds(row0 + j * _CHUNK, _CHUNK)])


def kernel(x, tables):
    b, f = x.shape
    nf, vocab, d = tables.shape
    rows = b * f
    rows_per_w = rows // _NW
    nchunk = rows_per_w // _CHUNK
    flat_idx = x.astype(jnp.int32) + jnp.arange(nf, dtype=jnp.int32)[None, :] * vocab
    flat_idx = flat_idx.reshape(_NW, nchunk, _CHUNK)
    tab = tables.reshape(nf * vocab, d)
    out = pl.kernel(
        _body,
        out_type=jax.ShapeDtypeStruct((rows, d), jnp.float32),
        mesh=plsc.VectorSubcoreMesh(core_axis_name="c", subcore_axis_name="s"),
        scratch_types=[
            pltpu.VMEM((nchunk, _CHUNK), jnp.int32),
            pltpu.VMEM((_CHUNK, d), jnp.float32),
            pltpu.SemaphoreType.DMA,
        ],
    )(flat_idx, tab)
    return out.reshape(b, f * d)


# speedup vs baseline: 3.2709x; 1.1849x over previous
import jax
import jax.numpy as jnp
from jax import lax
from jax.experimental import pallas as pl
from jax.experimental.pallas import tpu as pltpu
from jax.experimental.pallas import tpu_sc as plsc

_NC = 2
_NS = 16
_NW = _NC * _NS
_CHUNK = 128
_K = 2
_GROUP = _K * _CHUNK


def _body(idx_hbm, tab_hbm, out_hbm, idx_v, rows_v, gsem0, gsem1, ssem0, ssem1):
    wid = lax.axis_index("s") * _NC + lax.axis_index("c")
    nchunk = idx_v.shape[0]
    ngroup = nchunk // _K
    npairs = (ngroup - 2) // 2
    pltpu.sync_copy(idx_hbm.at[wid], idx_v)
    row0 = wid * nchunk * _CHUNK

    def gathers(g, buf, sem):
        for k in range(_K):
            pltpu.async_copy(tab_hbm.at[idx_v.at[g * _K + k]],
                             buf.at[pl.ds(k * _CHUNK, _CHUNK)], sem)

    def out_slab(g):
        return out_hbm.at[pl.ds(row0 + g * _GROUP, _GROUP)]

    def wait_gathers(buf, sem):
        pltpu.make_async_copy(out_hbm.at[pl.ds(0, _GROUP)], buf, sem).wait()

    def wait_scatter(buf, g, sem):
        pltpu.make_async_copy(buf, out_slab(g), sem).wait()

    buf0, buf1 = rows_v.at[0], rows_v.at[1]

    gathers(0, buf0, gsem0)
    wait_gathers(buf0, gsem0)
    pltpu.async_copy(buf0, out_slab(0), ssem0)
    gathers(1, buf1, gsem1)

    @pl.loop(0, npairs)
    def _pair(p):
        g = 2 * p + 1
        wait_gathers(buf1, gsem1)
        pltpu.async_copy(buf1, out_slab(g), ssem1)
        wait_scatter(buf0, g - 1, ssem0)
        gathers(g + 1, buf0, gsem0)
        wait_gathers(buf0, gsem0)
        pltpu.async_copy(buf0, out_slab(g + 1), ssem0)
        wait_scatter(buf1, g, ssem1)
        gathers(g + 2, buf1, gsem1)

    g_last = ngroup - 1
    wait_gathers(buf1, gsem1)
    pltpu.async_copy(buf1, out_slab(g_last), ssem1)
    wait_scatter(buf0, g_last - 1, ssem0)
    wait_scatter(buf1, g_last, ssem1)


def kernel(x, tables):
    b, f = x.shape
    nf, vocab, d = tables.shape
    rows = b * f
    rows_per_w = rows // _NW
    nchunk = rows_per_w // _CHUNK
    flat_idx = x.astype(jnp.int32) + jnp.arange(nf, dtype=jnp.int32)[None, :] * vocab
    flat_idx = flat_idx.reshape(_NW, nchunk, _CHUNK)
    tab = tables.reshape(nf * vocab, d)
    out = pl.kernel(
        _body,
        out_type=jax.ShapeDtypeStruct((rows, d), jnp.float32),
        mesh=plsc.VectorSubcoreMesh(core_axis_name="c", subcore_axis_name="s"),
        scratch_types=[
            pltpu.VMEM((nchunk, _CHUNK), jnp.int32),
            pltpu.VMEM((2, _GROUP, d), jnp.float32),
            pltpu.SemaphoreType.DMA,
            pltpu.SemaphoreType.DMA,
            pltpu.SemaphoreType.DMA,
            pltpu.SemaphoreType.DMA,
        ],
    )(flat_idx, tab)
    return out.reshape(b, f * d)


# speedup vs baseline: 6.1797x; 1.8893x over previous
import jax
import jax.numpy as jnp
from jax import lax
from jax.experimental import pallas as pl
from jax.experimental.pallas import tpu as pltpu
from jax.experimental.pallas import tpu_sc as plsc

_NC = 2
_NS = 16
_NW = _NC * _NS
_BB = 128


def _body(idx_hbm, tab_hbm, out_hbm, idx_v, rows_v, gsem0, gsem1, ssem0, ssem1):
    nf = out_hbm.shape[1] // 128
    wid = lax.axis_index("s") * _NC + lax.axis_index("c")
    nunit = idx_v.shape[0]
    u0 = wid * nunit
    pltpu.sync_copy(idx_hbm.at[wid], idx_v)

    def gather(u, buf, sem):
        pltpu.async_copy(tab_hbm.at[idx_v.at[u]], buf, sem)

    def out_slab(u):
        ug = u0 + u
        bb = ug // nf
        f = ug - bb * nf
        return out_hbm.at[pl.ds(bb * _BB, _BB), pl.ds(f * 128, 128)]

    def wait_gather(buf, sem):
        pltpu.make_async_copy(tab_hbm.at[pl.ds(0, _BB)], buf, sem).wait()

    def scatter(u, buf, sem):
        pltpu.async_copy(buf, out_slab(u), sem)

    def wait_scatter(buf, u, sem):
        pltpu.make_async_copy(buf, out_slab(u), sem).wait()

    buf0, buf1 = rows_v.at[0], rows_v.at[1]

    gather(0, buf0, gsem0)
    wait_gather(buf0, gsem0)
    scatter(0, buf0, ssem0)
    gather(1, buf1, gsem1)

    @pl.loop(0, (nunit - 2) // 2)
    def _pair(p):
        u = 2 * p + 1
        wait_gather(buf1, gsem1)
        scatter(u, buf1, ssem1)
        wait_scatter(buf0, u - 1, ssem0)
        gather(u + 1, buf0, gsem0)
        wait_gather(buf0, gsem0)
        scatter(u + 1, buf0, ssem0)
        wait_scatter(buf1, u, ssem1)
        gather(u + 2, buf1, gsem1)

    u_last = nunit - 1
    wait_gather(buf1, gsem1)
    scatter(u_last, buf1, ssem1)
    wait_scatter(buf0, u_last - 1, ssem0)
    wait_scatter(buf1, u_last, ssem1)


def kernel(x, tables):
    b, f = x.shape
    nf, vocab, d = tables.shape
    nbb = b // _BB
    nunits = nbb * nf
    units_per_w = nunits // _NW
    flat_idx = (x.astype(jnp.int32)
                + jnp.arange(nf, dtype=jnp.int32)[None, :] * vocab)
    flat_idx = flat_idx.reshape(nbb, _BB, nf).transpose(0, 2, 1)
    flat_idx = flat_idx.reshape(_NW, units_per_w, _BB)
    tab = tables.reshape(nf * vocab, d)
    out = pl.kernel(
        _body,
        out_type=jax.ShapeDtypeStruct((b, f * d), jnp.float32),
        mesh=plsc.VectorSubcoreMesh(core_axis_name="c", subcore_axis_name="s"),
        scratch_types=[
            pltpu.VMEM((units_per_w, _BB), jnp.int32),
            pltpu.VMEM((2, _BB, d), jnp.float32),
            pltpu.SemaphoreType.DMA,
            pltpu.SemaphoreType.DMA,
            pltpu.SemaphoreType.DMA,
            pltpu.SemaphoreType.DMA,
        ],
    )(flat_idx, tab)
    return out


# speedup vs baseline: 6.2183x; 1.0063x over previous
import jax
import jax.numpy as jnp
from jax import lax
from jax.experimental import pallas as pl
from jax.experimental.pallas import tpu as pltpu
from jax.experimental.pallas import tpu_sc as plsc

_NC = 2
_NS = 16
_NW = _NC * _NS
_BB = 128


def _body(idx_hbm, tab_hbm, out_hbm, idx_v, rows_v, gsem0, gsem1, ssem0, ssem1):
    nf = out_hbm.shape[1] // 128
    wid = lax.axis_index("s") * _NC + lax.axis_index("c")
    nunit = idx_v.shape[0]
    u0 = wid * nunit
    pltpu.sync_copy(idx_hbm.at[wid], idx_v)

    def gather(u, buf, sem):
        pltpu.async_copy(tab_hbm.at[idx_v.at[u]], buf, sem)

    def out_slab(u):
        ug = u0 + u
        bb = ug // nf
        f = ug - bb * nf
        return out_hbm.at[pl.ds(bb * _BB, _BB), pl.ds(f * 128, 128)]

    def wait_gather(buf, sem):
        pltpu.make_async_copy(tab_hbm.at[pl.ds(0, _BB)], buf, sem).wait()

    def scatter(u, buf, sem):
        pltpu.async_copy(buf, out_slab(u), sem)

    def wait_scatter(buf, u, sem):
        pltpu.make_async_copy(buf, out_slab(u), sem).wait()

    buf0, buf1 = rows_v.at[0], rows_v.at[1]

    gather(0, buf0, gsem0)
    wait_gather(buf0, gsem0)
    scatter(0, buf0, ssem0)
    gather(1, buf1, gsem1)

    @pl.loop(0, (nunit - 2) // 2)
    def _pair(p):
        u = 2 * p + 1
        wait_gather(buf1, gsem1)
        scatter(u, buf1, ssem1)
        wait_scatter(buf0, u - 1, ssem0)
        gather(u + 1, buf0, gsem0)
        wait_gather(buf0, gsem0)
        scatter(u + 1, buf0, ssem0)
        wait_scatter(buf1, u, ssem1)
        gather(u + 2, buf1, gsem1)

    u_last = nunit - 1
    wait_gather(buf1, gsem1)
    scatter(u_last, buf1, ssem1)
    wait_scatter(buf0, u_last - 1, ssem0)
    wait_scatter(buf1, u_last, ssem1)


def kernel(x, tables):
    b, f = x.shape
    nf, vocab, d = tables.shape
    nbb = b // _BB
    nunits = nbb * nf
    units_per_w = nunits // _NW
    flat_idx = (x.astype(jnp.int32)
                + jnp.arange(nf, dtype=jnp.int32)[None, :] * vocab)
    flat_idx = flat_idx.reshape(nbb, _BB, nf).transpose(0, 2, 1)
    flat_idx = flat_idx.reshape(_NW, units_per_w, _BB)
    tab = tables.reshape(nf * vocab, d)
    out = pl.kernel(
        _body,
        out_type=jax.ShapeDtypeStruct((b, f * d), jnp.float32),
        mesh=plsc.VectorSubcoreMesh(core_axis_name="c", subcore_axis_name="s"),
        compiler_params=pltpu.CompilerParams(use_tc_tiling_on_sc=True),
        scratch_types=[
            pltpu.VMEM((units_per_w, _BB), jnp.int32),
            pltpu.VMEM((2, _BB, d), jnp.float32),
            pltpu.SemaphoreType.DMA,
            pltpu.SemaphoreType.DMA,
            pltpu.SemaphoreType.DMA,
            pltpu.SemaphoreType.DMA,
        ],
    )(flat_idx, tab)
    return out
